# baseline (speedup 1.0000x reference)
import functools

import jax
import jax.numpy as jnp
from jax import lax
from jax.experimental import pallas as pl
from jax.experimental.pallas import tpu as pltpu
from jax.experimental.pallas import tpu_sc as plsc

N = 10000
E = 320000
D = 128
DZ = 64
NC, NS = 2, 16
NW = NC * NS
CH = 128
NCHUNK = 80
EP = NW * NCHUNK * CH
NPAD = N + 128
RPS = 624
NREM = N - RPS * NS

_sc_mesh = plsc.VectorSubcoreMesh(core_axis_name="c", subcore_axis_name="s")


@functools.partial(
    pl.kernel,
    out_type=jax.ShapeDtypeStruct((NC, 2, N, 16), jnp.float32),
    mesh=_sc_mesh,
    compiler_params=pltpu.CompilerParams(use_tc_tiling_on_sc=False),
    scratch_types=[
        pltpu.VMEM((CH,), jnp.int32),
        pltpu.VMEM((CH,), jnp.int32),
        pltpu.VMEM((CH,), jnp.int32),
        pltpu.VMEM((CH,), jnp.int32),
        pltpu.VMEM((CH, 16), jnp.float32),
        pltpu.VMEM_SHARED((NPAD, 16), jnp.float32),
        pltpu.VMEM_SHARED((NPAD, 16), jnp.float32),
        pltpu.SemaphoreType.DMA,
        pltpu.SemaphoreType.DMA,
        pltpu.SemaphoreType.DMA,
        pltpu.SemaphoreType.DMA,
    ],
)
def _deg_kernel(srcf_hbm, dstf_hbm, zeros16_hbm, ones16_hbm, out_hbm,
                idx_s0, idx_s1, idx_d0, idx_d1, ones_v, acc_s, acc_d,
                sem_l0, sem_l1, sem_c0, sem_c1):
    cid = lax.axis_index("c")
    sid = lax.axis_index("s")
    wid = cid * NS + sid
    base = wid * (NCHUNK * CH)

    pltpu.sync_copy(ones16_hbm, ones_v)
    for acc in (acc_s, acc_d):
        pltpu.sync_copy(zeros16_hbm, acc.at[pl.ds(sid * RPS, RPS)])
        @pl.when(sid == NS - 1)
        def _(acc=acc):
            pltpu.sync_copy(zeros16_hbm.at[pl.ds(0, NREM)],
                            acc.at[pl.ds(RPS * NS, NREM)])
    plsc.subcore_barrier()

    bufs = ((idx_s0, idx_d0, sem_l0, sem_c0),
            (idx_s1, idx_d1, sem_l1, sem_c1))

    def load(c, b):
        i_s, i_d, s_l, _ = bufs[b]
        eb = base + c * CH
        pltpu.async_copy(srcf_hbm.at[pl.ds(eb, CH)], i_s, s_l)
        pltpu.async_copy(dstf_hbm.at[pl.ds(eb, CH)], i_d, s_l)

    def wait_load(b):
        i_s, i_d, s_l, _ = bufs[b]
        pltpu.make_async_copy(srcf_hbm.at[pl.ds(0, CH)], i_s, s_l).wait()
        pltpu.make_async_copy(dstf_hbm.at[pl.ds(0, CH)], i_d, s_l).wait()

    def scat(b):
        i_s, i_d, _, s_c = bufs[b]
        pltpu.async_copy(ones_v, acc_s.at[i_s], s_c, add=True)
        pltpu.async_copy(ones_v, acc_d.at[i_d], s_c, add=True)

    def wait_scat(b):
        i_s, i_d, _, s_c = bufs[b]
        pltpu.make_async_copy(ones_v, acc_s.at[i_s], s_c).wait()
        pltpu.make_async_copy(ones_v, acc_d.at[i_d], s_c).wait()

    load(0, 0)

    def body(c, _):
        for p in (0, 1):
            @pl.when(c % 2 == p)
            def _(p=p):
                wait_load(p)
                scat(p)
                @pl.when(c > 0)
                def _():
                    wait_scat(1 - p)
                @pl.when(c + 1 < NCHUNK)
                def _():
                    load(c + 1, 1 - p)
        return 0
    lax.fori_loop(0, NCHUNK, body, 0)
    wait_scat((NCHUNK - 1) % 2)
    plsc.subcore_barrier()

    sl = pl.ds(sid * RPS, RPS)
    pltpu.sync_copy(acc_s.at[sl], out_hbm.at[cid, 0, sl])
    pltpu.sync_copy(acc_d.at[sl], out_hbm.at[cid, 1, sl])
    @pl.when(sid == NS - 1)
    def _():
        rem = pl.ds(RPS * NS, NREM)
        pltpu.sync_copy(acc_s.at[rem], out_hbm.at[cid, 0, rem])
        pltpu.sync_copy(acc_d.at[rem], out_hbm.at[cid, 1, rem])


@functools.partial(
    pl.kernel,
    out_type=jax.ShapeDtypeStruct((NC, N, D), jnp.float32),
    mesh=_sc_mesh,
    scratch_types=[
        pltpu.VMEM((CH,), jnp.int32),
        pltpu.VMEM((CH,), jnp.int32),
        pltpu.VMEM((CH,), jnp.int32),
        pltpu.VMEM((CH,), jnp.int32),
        pltpu.VMEM((CH, D), jnp.float32),
        pltpu.VMEM((CH, D), jnp.float32),
        pltpu.VMEM_SHARED((NPAD, D), jnp.float32),
        pltpu.SemaphoreType.DMA,
        pltpu.SemaphoreType.DMA,
        pltpu.SemaphoreType.DMA,
        pltpu.SemaphoreType.DMA,
    ],
)
def _agg_kernel(y_hbm, srcf_hbm, dstf_hbm, zeros_hbm, out_hbm,
                idx_s0, idx_s1, idx_d0, idx_d1, rows0, rows1, acc,
                sem_l0, sem_l1, sem_g0, sem_g1):
    cid = lax.axis_index("c")
    sid = lax.axis_index("s")
    wid = cid * NS + sid
    base = wid * (NCHUNK * CH)

    pltpu.sync_copy(zeros_hbm, acc.at[pl.ds(sid * RPS, RPS)])
    @pl.when(sid == NS - 1)
    def _():
        pltpu.sync_copy(zeros_hbm.at[pl.ds(0, NREM)],
                        acc.at[pl.ds(RPS * NS, NREM)])

    bufs = ((idx_s0, idx_d0, sem_l0, rows0, sem_g0),
            (idx_s1, idx_d1, sem_l1, rows1, sem_g1))

    def load(c, b):
        i_s, i_d, s_l, _, _ = bufs[b]
        eb = base + c * CH
        pltpu.async_copy(srcf_hbm.at[pl.ds(eb, CH)], i_s, s_l)
        pltpu.async_copy(dstf_hbm.at[pl.ds(eb, CH)], i_d, s_l)

    def wait_load(b):
        i_s, i_d, s_l, _, _ = bufs[b]
        pltpu.make_async_copy(srcf_hbm.at[pl.ds(0, CH)], i_s, s_l).wait()
        pltpu.make_async_copy(dstf_hbm.at[pl.ds(0, CH)], i_d, s_l).wait()

    def gather(b):
        i_s, _, _, r, s_g = bufs[b]
        pltpu.async_copy(y_hbm.at[i_s], r, s_g)

    def wait_gather(b):
        i_s, _, _, r, s_g = bufs[b]
        pltpu.make_async_copy(y_hbm.at[i_s], r, s_g).wait()

    load(0, 0)
    wait_load(0)
    gather(0)
    load(1, 1)
    plsc.subcore_barrier()

    def body(c, _):
        for p in (0, 1):
            @pl.when(c % 2 == p)
            def _(p=p):
                @pl.when(c + 1 < NCHUNK)
                def _():
                    wait_load(1 - p)
                    gather(1 - p)
                wait_gather(p)
                i_s, i_d, _, r, _sg = bufs[p]
                pltpu.sync_copy(r, acc.at[i_d], add=True)
                @pl.when(c + 2 < NCHUNK)
                def _():
                    load(c + 2, p)
        return 0
    lax.fori_loop(0, NCHUNK, body, 0)
    plsc.subcore_barrier()

    sl = pl.ds(sid * RPS, RPS)
    pltpu.sync_copy(acc.at[sl], out_hbm.at[cid, sl])
    @pl.when(sid == NS - 1)
    def _():
        rem = pl.ds(RPS * NS, NREM)
        pltpu.sync_copy(acc.at[rem], out_hbm.at[cid, rem])


_RB = 1000
_GRID = N // _RB

_deg_spec = pl.BlockSpec((NC, 2, _RB, 16), lambda r: (0, 0, r, 0))


def _norms(degref):
    deg_out = degref[0, 0, :, :1] + degref[1, 0, :, :1]
    deg_in = degref[0, 1, :, :1] + degref[1, 1, :, :1]
    n_out = lax.rsqrt(jnp.maximum(deg_out, 1.0))
    n_in = lax.rsqrt(jnp.maximum(deg_in, 1.0))
    return n_out, n_in


def _tc1_body(x_ref, w_ref, deg_ref, y_ref):
    n_out, _ = _norms(deg_ref)
    xw = jnp.dot(x_ref[...], w_ref[...], preferred_element_type=jnp.float32)
    y_ref[...] = xw * n_out


_tc1 = pl.pallas_call(
    _tc1_body,
    grid=(_GRID,),
    in_specs=[
        pl.BlockSpec((_RB, D), lambda r: (r, 0)),
        pl.BlockSpec((D, D), lambda r: (0, 0)),
        _deg_spec,
    ],
    out_specs=pl.BlockSpec((_RB, D), lambda r: (r, 0)),
    out_shape=jax.ShapeDtypeStruct((N, D), jnp.float32),
)


def _tc2_body(p_ref, deg_ref, y_ref):
    n_out, n_in = _norms(deg_ref)
    h = jnp.maximum((p_ref[0] + p_ref[1]) * n_in, 0.0)
    y_ref[...] = h * n_out


_tc2 = pl.pallas_call(
    _tc2_body,
    grid=(_GRID,),
    in_specs=[
        pl.BlockSpec((NC, _RB, D), lambda r: (0, r, 0)),
        _deg_spec,
    ],
    out_specs=pl.BlockSpec((_RB, D), lambda r: (r, 0)),
    out_shape=jax.ShapeDtypeStruct((N, D), jnp.float32),
)


def _tc3_body(p_ref, deg_ref, wmu_ref, wlv_ref, eps_ref, z_ref, mu_ref,
              lv_ref):
    _, n_in = _norms(deg_ref)
    agg = (p_ref[0] + p_ref[1]) * n_in
    mu = jnp.dot(agg, wmu_ref[...], preferred_element_type=jnp.float32)
    lv = jnp.dot(agg, wlv_ref[...], preferred_element_type=jnp.float32)
    mu_ref[...] = mu
    lv_ref[...] = lv
    z_ref[...] = mu + jnp.exp(0.5 * lv) * eps_ref[...]


_tc3 = pl.pallas_call(
    _tc3_body,
    grid=(_GRID,),
    in_specs=[
        pl.BlockSpec((NC, _RB, D), lambda r: (0, r, 0)),
        _deg_spec,
        pl.BlockSpec((D, DZ), lambda r: (0, 0)),
        pl.BlockSpec((D, DZ), lambda r: (0, 0)),
        pl.BlockSpec((_RB, DZ), lambda r: (r, 0)),
    ],
    out_specs=[
        pl.BlockSpec((_RB, DZ), lambda r: (r, 0)),
        pl.BlockSpec((_RB, DZ), lambda r: (r, 0)),
        pl.BlockSpec((_RB, DZ), lambda r: (r, 0)),
    ],
    out_shape=[
        jax.ShapeDtypeStruct((N, DZ), jnp.float32),
        jax.ShapeDtypeStruct((N, DZ), jnp.float32),
        jax.ShapeDtypeStruct((N, DZ), jnp.float32),
    ],
)


def kernel(x, edge_index, W1, W_mu, W_logvar):
    src = edge_index[0]
    dst = edge_index[1]
    npad = EP - E
    pad_n = N + (jnp.arange(npad, dtype=jnp.int32) % (NPAD - N))
    srcp = jnp.concatenate([src, pad_n])
    dstp = jnp.concatenate([dst, pad_n])
    srca = jnp.concatenate([src, jnp.zeros((npad,), jnp.int32)])
    zeros16 = jnp.zeros((RPS, 16), jnp.float32)
    ones16 = jnp.ones((CH, 16), jnp.float32)
    zeros128 = jnp.zeros((RPS, D), jnp.float32)

    degs = _deg_kernel(srcp, dstp, zeros16, ones16)
    y1 = _tc1(x, W1, degs)
    p1 = _agg_kernel(y1, srca, dstp, zeros128)
    y2 = _tc2(p1, degs)
    p2 = _agg_kernel(y2, srca, dstp, zeros128)
    eps = jax.random.normal(jax.random.key(42), (N, DZ), dtype=jnp.float32)
    z, mu, lv = _tc3(p2, degs, W_mu, W_logvar, eps)
    return z, mu, lv


# speedup vs baseline: 1.9346x; 1.9346x over previous
import functools

import jax
import jax.numpy as jnp
from jax import lax
from jax.experimental import pallas as pl
from jax.experimental.pallas import tpu as pltpu
from jax.experimental.pallas import tpu_sc as plsc

N = 10000
E = 320000
D = 128
DH = D // 2
DZ = 64
NC, NS = 2, 16
NW = NC * NS
CH = 128
NCHUNK = 80
EP = NW * NCHUNK * CH
NCH_ALL = EP // CH
CPT = NCH_ALL // NS
NPAD = N + 128
RPS = 624
NREM = N - RPS * NS

_sc_mesh = plsc.VectorSubcoreMesh(core_axis_name="c", subcore_axis_name="s")


@functools.partial(
    pl.kernel,
    out_type=jax.ShapeDtypeStruct((NC, 2, N, 16), jnp.float32),
    mesh=_sc_mesh,
    compiler_params=pltpu.CompilerParams(use_tc_tiling_on_sc=False),
    scratch_types=[
        pltpu.VMEM((CH,), jnp.int32),
        pltpu.VMEM((CH,), jnp.int32),
        pltpu.VMEM((CH,), jnp.int32),
        pltpu.VMEM((CH,), jnp.int32),
        pltpu.VMEM((CH, 16), jnp.float32),
        pltpu.VMEM_SHARED((NPAD, 16), jnp.float32),
        pltpu.VMEM_SHARED((NPAD, 16), jnp.float32),
        pltpu.SemaphoreType.DMA,
        pltpu.SemaphoreType.DMA,
        pltpu.SemaphoreType.DMA,
        pltpu.SemaphoreType.DMA,
    ],
)
def _deg_kernel(srcf_hbm, dstf_hbm, zeros16_hbm, ones16_hbm, out_hbm,
                idx_s0, idx_s1, idx_d0, idx_d1, ones_v, acc_s, acc_d,
                sem_l0, sem_l1, sem_c0, sem_c1):
    cid = lax.axis_index("c")
    sid = lax.axis_index("s")
    wid = cid * NS + sid
    base = wid * (NCHUNK * CH)

    pltpu.sync_copy(ones16_hbm, ones_v)
    for acc in (acc_s, acc_d):
        pltpu.sync_copy(zeros16_hbm, acc.at[pl.ds(sid * RPS, RPS)])
        @pl.when(sid == NS - 1)
        def _(acc=acc):
            pltpu.sync_copy(zeros16_hbm.at[pl.ds(0, NREM)],
                            acc.at[pl.ds(RPS * NS, NREM)])
    plsc.subcore_barrier()

    bufs = ((idx_s0, idx_d0, sem_l0, sem_c0),
            (idx_s1, idx_d1, sem_l1, sem_c1))

    def load(c, b):
        i_s, i_d, s_l, _ = bufs[b]
        eb = base + c * CH
        pltpu.async_copy(srcf_hbm.at[pl.ds(eb, CH)], i_s, s_l)
        pltpu.async_copy(dstf_hbm.at[pl.ds(eb, CH)], i_d, s_l)

    def wait_load(b):
        i_s, i_d, s_l, _ = bufs[b]
        pltpu.make_async_copy(srcf_hbm.at[pl.ds(0, CH)], i_s, s_l).wait()
        pltpu.make_async_copy(dstf_hbm.at[pl.ds(0, CH)], i_d, s_l).wait()

    def scat(b):
        i_s, i_d, _, s_c = bufs[b]
        pltpu.async_copy(ones_v, acc_s.at[i_s], s_c, add=True)
        pltpu.async_copy(ones_v, acc_d.at[i_d], s_c, add=True)

    def wait_scat(b):
        i_s, i_d, _, s_c = bufs[b]
        pltpu.make_async_copy(ones_v, acc_s.at[i_s], s_c).wait()
        pltpu.make_async_copy(ones_v, acc_d.at[i_d], s_c).wait()

    load(0, 0)

    def body(c, _):
        for p in (0, 1):
            @pl.when(c % 2 == p)
            def _(p=p):
                wait_load(p)
                scat(p)
                @pl.when(c > 0)
                def _():
                    wait_scat(1 - p)
                @pl.when(c + 1 < NCHUNK)
                def _():
                    load(c + 1, 1 - p)
        return 0
    lax.fori_loop(0, NCHUNK, body, 0)
    wait_scat((NCHUNK - 1) % 2)
    plsc.subcore_barrier()

    sl = pl.ds(sid * RPS, RPS)
    pltpu.sync_copy(acc_s.at[sl], out_hbm.at[cid, 0, sl])
    pltpu.sync_copy(acc_d.at[sl], out_hbm.at[cid, 1, sl])
    @pl.when(sid == NS - 1)
    def _():
        rem = pl.ds(RPS * NS, NREM)
        pltpu.sync_copy(acc_s.at[rem], out_hbm.at[cid, 0, rem])
        pltpu.sync_copy(acc_d.at[rem], out_hbm.at[cid, 1, rem])


@functools.partial(
    pl.kernel,
    out_type=jax.ShapeDtypeStruct((NC, N, DH), jnp.float32),
    mesh=_sc_mesh,
    compiler_params=pltpu.CompilerParams(use_tc_tiling_on_sc=False),
    scratch_types=[
        pltpu.VMEM((CH,), jnp.int32),
        pltpu.VMEM((CH,), jnp.int32),
        pltpu.VMEM((CH,), jnp.int32),
        pltpu.VMEM((CH,), jnp.int32),
        pltpu.VMEM((CH, DH), jnp.float32),
        pltpu.VMEM((CH, DH), jnp.float32),
        pltpu.VMEM_SHARED((N, DH), jnp.float32),
        pltpu.VMEM_SHARED((NPAD, DH), jnp.float32),
        pltpu.SemaphoreType.DMA,
        pltpu.SemaphoreType.DMA,
        pltpu.SemaphoreType.DMA,
        pltpu.SemaphoreType.DMA,
    ],
)
def _agg_kernel(y2_hbm, srcf_hbm, dstf_hbm, zeros_hbm, out_hbm,
                idx_s0, idx_s1, idx_d0, idx_d1, rows0, rows1, ys, acc,
                sem_l0, sem_l1, sem_g0, sem_g1):
    cid = lax.axis_index("c")
    sid = lax.axis_index("s")
    base = sid * (CPT * CH)

    sl = pl.ds(sid * RPS, RPS)
    pltpu.sync_copy(y2_hbm.at[cid, sl], ys.at[sl])
    pltpu.sync_copy(zeros_hbm, acc.at[sl])
    @pl.when(sid == NS - 1)
    def _():
        rem = pl.ds(RPS * NS, NREM)
        pltpu.sync_copy(y2_hbm.at[cid, rem], ys.at[rem])
        pltpu.sync_copy(zeros_hbm.at[pl.ds(0, NREM)], acc.at[rem])

    bufs = ((idx_s0, idx_d0, sem_l0, rows0, sem_g0),
            (idx_s1, idx_d1, sem_l1, rows1, sem_g1))

    def load(c, b):
        i_s, i_d, s_l, _, _ = bufs[b]
        eb = base + c * CH
        pltpu.async_copy(srcf_hbm.at[pl.ds(eb, CH)], i_s, s_l)
        pltpu.async_copy(dstf_hbm.at[pl.ds(eb, CH)], i_d, s_l)

    def wait_load(b):
        i_s, i_d, s_l, _, _ = bufs[b]
        pltpu.make_async_copy(srcf_hbm.at[pl.ds(0, CH)], i_s, s_l).wait()
        pltpu.make_async_copy(dstf_hbm.at[pl.ds(0, CH)], i_d, s_l).wait()

    def gather(b):
        i_s, _, _, r, s_g = bufs[b]
        pltpu.async_copy(ys.at[i_s], r, s_g)

    def wait_gather(b):
        i_s, _, _, r, s_g = bufs[b]
        pltpu.make_async_copy(ys.at[i_s], r, s_g).wait()

    load(0, 0)
    wait_load(0)
    load(1, 1)
    plsc.subcore_barrier()
    gather(0)

    def body(c, _):
        for p in (0, 1):
            @pl.when(c % 2 == p)
            def _(p=p):
                @pl.when(c + 1 < CPT)
                def _():
                    wait_load(1 - p)
                    gather(1 - p)
                wait_gather(p)
                i_s, i_d, _, r, _sg = bufs[p]
                pltpu.sync_copy(r, acc.at[i_d], add=True)
                @pl.when(c + 2 < CPT)
                def _():
                    load(c + 2, p)
        return 0
    lax.fori_loop(0, CPT, body, 0)
    plsc.subcore_barrier()

    pltpu.sync_copy(acc.at[sl], out_hbm.at[cid, sl])
    @pl.when(sid == NS - 1)
    def _():
        rem = pl.ds(RPS * NS, NREM)
        pltpu.sync_copy(acc.at[rem], out_hbm.at[cid, rem])


_RB = 1000
_GRID = N // _RB

_deg_spec = pl.BlockSpec((NC, 2, _RB, 16), lambda r: (0, 0, r, 0))
_half_spec = pl.BlockSpec((2, _RB, DH), lambda r: (0, r, 0))


def _norms(degref):
    deg_out = degref[0, 0, :, :1] + degref[1, 0, :, :1]
    deg_in = degref[0, 1, :, :1] + degref[1, 1, :, :1]
    n_out = lax.rsqrt(jnp.maximum(deg_out, 1.0))
    n_in = lax.rsqrt(jnp.maximum(deg_in, 1.0))
    return n_out, n_in


def _tc1_body(x_ref, w_ref, deg_ref, y_ref):
    n_out, _ = _norms(deg_ref)
    xv = x_ref[...]
    y_ref[0] = jnp.dot(xv, w_ref[:, :DH],
                       preferred_element_type=jnp.float32) * n_out
    y_ref[1] = jnp.dot(xv, w_ref[:, DH:],
                       preferred_element_type=jnp.float32) * n_out


_tc1 = pl.pallas_call(
    _tc1_body,
    grid=(_GRID,),
    in_specs=[
        pl.BlockSpec((_RB, D), lambda r: (r, 0)),
        pl.BlockSpec((D, D), lambda r: (0, 0)),
        _deg_spec,
    ],
    out_specs=_half_spec,
    out_shape=jax.ShapeDtypeStruct((2, N, DH), jnp.float32),
)


def _tc2_body(p_ref, deg_ref, y_ref):
    n_out, n_in = _norms(deg_ref)
    for i in range(2):
        h = jnp.maximum(p_ref[i] * n_in, 0.0)
        y_ref[i] = h * n_out


_tc2 = pl.pallas_call(
    _tc2_body,
    grid=(_GRID,),
    in_specs=[_half_spec, _deg_spec],
    out_specs=_half_spec,
    out_shape=jax.ShapeDtypeStruct((2, N, DH), jnp.float32),
)


def _tc3_body(p_ref, deg_ref, wmu_ref, wlv_ref, eps_ref, z_ref, mu_ref,
              lv_ref):
    _, n_in = _norms(deg_ref)
    a_lo = p_ref[0] * n_in
    a_hi = p_ref[1] * n_in
    mu = (jnp.dot(a_lo, wmu_ref[:DH, :], preferred_element_type=jnp.float32)
          + jnp.dot(a_hi, wmu_ref[DH:, :], preferred_element_type=jnp.float32))
    lv = (jnp.dot(a_lo, wlv_ref[:DH, :], preferred_element_type=jnp.float32)
          + jnp.dot(a_hi, wlv_ref[DH:, :], preferred_element_type=jnp.float32))
    mu_ref[...] = mu
    lv_ref[...] = lv
    z_ref[...] = mu + jnp.exp(0.5 * lv) * eps_ref[...]


_tc3 = pl.pallas_call(
    _tc3_body,
    grid=(_GRID,),
    in_specs=[
        _half_spec,
        _deg_spec,
        pl.BlockSpec((D, DZ), lambda r: (0, 0)),
        pl.BlockSpec((D, DZ), lambda r: (0, 0)),
        pl.BlockSpec((_RB, DZ), lambda r: (r, 0)),
    ],
    out_specs=[
        pl.BlockSpec((_RB, DZ), lambda r: (r, 0)),
        pl.BlockSpec((_RB, DZ), lambda r: (r, 0)),
        pl.BlockSpec((_RB, DZ), lambda r: (r, 0)),
    ],
    out_shape=[
        jax.ShapeDtypeStruct((N, DZ), jnp.float32),
        jax.ShapeDtypeStruct((N, DZ), jnp.float32),
        jax.ShapeDtypeStruct((N, DZ), jnp.float32),
    ],
)


def kernel(x, edge_index, W1, W_mu, W_logvar):
    src = edge_index[0]
    dst = edge_index[1]
    npad = EP - E
    pad_n = N + (jnp.arange(npad, dtype=jnp.int32) % (NPAD - N))
    srcp = jnp.concatenate([src, pad_n])
    dstp = jnp.concatenate([dst, pad_n])
    srca = jnp.concatenate([src, jnp.zeros((npad,), jnp.int32)])
    zeros16 = jnp.zeros((RPS, 16), jnp.float32)
    ones16 = jnp.ones((CH, 16), jnp.float32)
    zeros64 = jnp.zeros((RPS, DH), jnp.float32)

    degs = _deg_kernel(srcp, dstp, zeros16, ones16)
    y1 = _tc1(x, W1, degs)
    p1 = _agg_kernel(y1, srca, dstp, zeros64)
    y2 = _tc2(p1, degs)
    p2 = _agg_kernel(y2, srca, dstp, zeros64)
    eps = jax.random.normal(jax.random.key(42), (N, DZ), dtype=jnp.float32)
    z, mu, lv = _tc3(p2, degs, W_mu, W_logvar, eps)
    return z, mu, lv


# speedup vs baseline: 2.3922x; 1.2366x over previous
import functools

import jax
import jax.numpy as jnp
from jax import lax
from jax.experimental import pallas as pl
from jax.experimental.pallas import tpu as pltpu
from jax.experimental.pallas import tpu_sc as plsc

N = 10000
E = 320000
D = 128
DH = D // 2
DZ = 64
NC, NS = 2, 16
NW = NC * NS
CH = 128
NCHUNK = 80
EP = NW * NCHUNK * CH
NCH_ALL = EP // CH
CPT = NCH_ALL // NS
NPAD = N + 128
RPS = 624
NREM = N - RPS * NS

_sc_mesh = plsc.VectorSubcoreMesh(core_axis_name="c", subcore_axis_name="s")


@functools.partial(
    pl.kernel,
    out_type=jax.ShapeDtypeStruct((NC, 2, N, 16), jnp.float32),
    mesh=_sc_mesh,
    compiler_params=pltpu.CompilerParams(use_tc_tiling_on_sc=False),
    scratch_types=[
        pltpu.VMEM((CH,), jnp.int32),
        pltpu.VMEM((CH,), jnp.int32),
        pltpu.VMEM((CH,), jnp.int32),
        pltpu.VMEM((CH,), jnp.int32),
        pltpu.VMEM((CH, 16), jnp.float32),
        pltpu.VMEM_SHARED((NPAD, 16), jnp.float32),
        pltpu.VMEM_SHARED((NPAD, 16), jnp.float32),
        pltpu.SemaphoreType.DMA,
        pltpu.SemaphoreType.DMA,
        pltpu.SemaphoreType.DMA,
        pltpu.SemaphoreType.DMA,
    ],
)
def _deg_kernel(srcf_hbm, dstf_hbm, zeros16_hbm, ones16_hbm, out_hbm,
                idx_s0, idx_s1, idx_d0, idx_d1, ones_v, acc_s, acc_d,
                sem_l0, sem_l1, sem_c0, sem_c1):
    cid = lax.axis_index("c")
    sid = lax.axis_index("s")
    wid = cid * NS + sid
    base = wid * (NCHUNK * CH)

    pltpu.sync_copy(ones16_hbm, ones_v)
    for acc in (acc_s, acc_d):
        pltpu.sync_copy(zeros16_hbm, acc.at[pl.ds(sid * RPS, RPS)])
        @pl.when(sid == NS - 1)
        def _(acc=acc):
            pltpu.sync_copy(zeros16_hbm.at[pl.ds(0, NREM)],
                            acc.at[pl.ds(RPS * NS, NREM)])
    plsc.subcore_barrier()

    bufs = ((idx_s0, idx_d0, sem_l0, sem_c0),
            (idx_s1, idx_d1, sem_l1, sem_c1))

    def load(c, b):
        i_s, i_d, s_l, _ = bufs[b]
        eb = base + c * CH
        pltpu.async_copy(srcf_hbm.at[pl.ds(eb, CH)], i_s, s_l)
        pltpu.async_copy(dstf_hbm.at[pl.ds(eb, CH)], i_d, s_l)

    def wait_load(b):
        i_s, i_d, s_l, _ = bufs[b]
        pltpu.make_async_copy(srcf_hbm.at[pl.ds(0, CH)], i_s, s_l).wait()
        pltpu.make_async_copy(dstf_hbm.at[pl.ds(0, CH)], i_d, s_l).wait()

    def scat(b):
        i_s, i_d, _, s_c = bufs[b]
        pltpu.async_copy(ones_v, acc_s.at[i_s], s_c, add=True)
        pltpu.async_copy(ones_v, acc_d.at[i_d], s_c, add=True)

    def wait_scat(b):
        i_s, i_d, _, s_c = bufs[b]
        pltpu.make_async_copy(ones_v, acc_s.at[i_s], s_c).wait()
        pltpu.make_async_copy(ones_v, acc_d.at[i_d], s_c).wait()

    load(0, 0)

    def body(c, _):
        for p in (0, 1):
            @pl.when(c % 2 == p)
            def _(p=p):
                wait_load(p)
                scat(p)
                @pl.when(c > 0)
                def _():
                    wait_scat(1 - p)
                @pl.when(c + 1 < NCHUNK)
                def _():
                    load(c + 1, 1 - p)
        return 0
    lax.fori_loop(0, NCHUNK, body, 0)
    wait_scat((NCHUNK - 1) % 2)
    plsc.subcore_barrier()

    sl = pl.ds(sid * RPS, RPS)
    pltpu.sync_copy(acc_s.at[sl], out_hbm.at[cid, 0, sl])
    pltpu.sync_copy(acc_d.at[sl], out_hbm.at[cid, 1, sl])
    @pl.when(sid == NS - 1)
    def _():
        rem = pl.ds(RPS * NS, NREM)
        pltpu.sync_copy(acc_s.at[rem], out_hbm.at[cid, 0, rem])
        pltpu.sync_copy(acc_d.at[rem], out_hbm.at[cid, 1, rem])


@functools.partial(
    pl.kernel,
    out_type=jax.ShapeDtypeStruct((NC, N, DH), jnp.float32),
    mesh=_sc_mesh,
    compiler_params=pltpu.CompilerParams(use_tc_tiling_on_sc=False),
    scratch_types=[
        pltpu.VMEM((CH,), jnp.int32),
        pltpu.VMEM((CH,), jnp.int32),
        pltpu.VMEM((4, CH), jnp.int32),
        pltpu.VMEM((CH, DH), jnp.float32),
        pltpu.VMEM((CH, DH), jnp.float32),
        pltpu.VMEM_SHARED((N, DH), jnp.float32),
        pltpu.VMEM_SHARED((NPAD, DH), jnp.float32),
        pltpu.SemaphoreType.DMA,
        pltpu.SemaphoreType.DMA,
        pltpu.SemaphoreType.DMA,
        pltpu.SemaphoreType.DMA,
        pltpu.SemaphoreType.DMA,
    ],
)
def _agg_kernel(y2_hbm, srcf_hbm, dstf_hbm, zeros_hbm, out_hbm,
                idx_s0, idx_s1, idx_d, rows0, rows1, ys, acc,
                sem_l0, sem_l1, sem_g0, sem_g1, sem_c):
    cid = lax.axis_index("c")
    sid = lax.axis_index("s")
    base = sid * (CPT * CH)

    sl = pl.ds(sid * RPS, RPS)
    pltpu.sync_copy(y2_hbm.at[cid, sl], ys.at[sl])
    pltpu.sync_copy(zeros_hbm, acc.at[sl])
    @pl.when(sid == NS - 1)
    def _():
        rem = pl.ds(RPS * NS, NREM)
        pltpu.sync_copy(y2_hbm.at[cid, rem], ys.at[rem])
        pltpu.sync_copy(zeros_hbm.at[pl.ds(0, NREM)], acc.at[rem])

    bufs = ((idx_s0, sem_l0, rows0, sem_g0),
            (idx_s1, sem_l1, rows1, sem_g1))

    def load(c, b):
        i_s, s_l, _, _ = bufs[b]
        eb = base + c * CH
        pltpu.async_copy(srcf_hbm.at[pl.ds(eb, CH)], i_s, s_l)
        pltpu.async_copy(dstf_hbm.at[pl.ds(eb, CH)], idx_d.at[c % 4], s_l)

    def wait_load(b):
        i_s, s_l, _, _ = bufs[b]
        pltpu.make_async_copy(srcf_hbm.at[pl.ds(0, CH)], i_s, s_l).wait()
        pltpu.make_async_copy(srcf_hbm.at[pl.ds(0, CH)], i_s, s_l).wait()

    def gather(b):
        i_s, _, r, s_g = bufs[b]
        pltpu.async_copy(ys.at[i_s], r, s_g)

    def wait_gather(b):
        i_s, _, r, s_g = bufs[b]
        pltpu.make_async_copy(ys.at[i_s], r, s_g).wait()

    def scat(c, b):
        _, _, r, _ = bufs[b]
        pltpu.async_copy(r, acc.at[idx_d.at[c % 4]], sem_c, add=True)

    def wait_scat(b):
        _, _, r, _ = bufs[b]
        pltpu.make_async_copy(r, acc.at[idx_d.at[0]], sem_c).wait()

    load(0, 0)
    wait_load(0)
    load(1, 1)
    plsc.subcore_barrier()
    gather(0)

    def body(c, _):
        for p in (0, 1):
            @pl.when(c % 2 == p)
            def _(p=p):
                @pl.when(c + 1 < CPT)
                def _():
                    wait_load(1 - p)
                @pl.when(c > 0)
                def _():
                    wait_scat(1 - p)
                @pl.when(c + 1 < CPT)
                def _():
                    gather(1 - p)
                wait_gather(p)
                scat(c, p)
                @pl.when(c + 2 < CPT)
                def _():
                    load(c + 2, p)
        return 0
    lax.fori_loop(0, CPT, body, 0)
    wait_scat((CPT - 1) % 2)
    plsc.subcore_barrier()

    pltpu.sync_copy(acc.at[sl], out_hbm.at[cid, sl])
    @pl.when(sid == NS - 1)
    def _():
        rem = pl.ds(RPS * NS, NREM)
        pltpu.sync_copy(acc.at[rem], out_hbm.at[cid, rem])


_RB = 1000
_GRID = N // _RB

_deg_spec = pl.BlockSpec((NC, 2, _RB, 16), lambda r: (0, 0, r, 0))
_half_spec = pl.BlockSpec((2, _RB, DH), lambda r: (0, r, 0))


def _norms(degref):
    deg_out = degref[0, 0, :, :1] + degref[1, 0, :, :1]
    deg_in = degref[0, 1, :, :1] + degref[1, 1, :, :1]
    n_out = lax.rsqrt(jnp.maximum(deg_out, 1.0))
    n_in = lax.rsqrt(jnp.maximum(deg_in, 1.0))
    return n_out, n_in


def _tc1_body(x_ref, w_ref, deg_ref, y_ref):
    n_out, _ = _norms(deg_ref)
    xv = x_ref[...]
    y_ref[0] = jnp.dot(xv, w_ref[:, :DH],
                       preferred_element_type=jnp.float32) * n_out
    y_ref[1] = jnp.dot(xv, w_ref[:, DH:],
                       preferred_element_type=jnp.float32) * n_out


_tc1 = pl.pallas_call(
    _tc1_body,
    grid=(_GRID,),
    in_specs=[
        pl.BlockSpec((_RB, D), lambda r: (r, 0)),
        pl.BlockSpec((D, D), lambda r: (0, 0)),
        _deg_spec,
    ],
    out_specs=_half_spec,
    out_shape=jax.ShapeDtypeStruct((2, N, DH), jnp.float32),
)


def _tc2_body(p_ref, deg_ref, y_ref):
    n_out, n_in = _norms(deg_ref)
    for i in range(2):
        h = jnp.maximum(p_ref[i] * n_in, 0.0)
        y_ref[i] = h * n_out


_tc2 = pl.pallas_call(
    _tc2_body,
    grid=(_GRID,),
    in_specs=[_half_spec, _deg_spec],
    out_specs=_half_spec,
    out_shape=jax.ShapeDtypeStruct((2, N, DH), jnp.float32),
)


def _tc3_body(p_ref, deg_ref, wmu_ref, wlv_ref, eps_ref, z_ref, mu_ref,
              lv_ref):
    _, n_in = _norms(deg_ref)
    a_lo = p_ref[0] * n_in
    a_hi = p_ref[1] * n_in
    mu = (jnp.dot(a_lo, wmu_ref[:DH, :], preferred_element_type=jnp.float32)
          + jnp.dot(a_hi, wmu_ref[DH:, :], preferred_element_type=jnp.float32))
    lv = (jnp.dot(a_lo, wlv_ref[:DH, :], preferred_element_type=jnp.float32)
          + jnp.dot(a_hi, wlv_ref[DH:, :], preferred_element_type=jnp.float32))
    mu_ref[...] = mu
    lv_ref[...] = lv
    z_ref[...] = mu + jnp.exp(0.5 * lv) * eps_ref[...]


_tc3 = pl.pallas_call(
    _tc3_body,
    grid=(_GRID,),
    in_specs=[
        _half_spec,
        _deg_spec,
        pl.BlockSpec((D, DZ), lambda r: (0, 0)),
        pl.BlockSpec((D, DZ), lambda r: (0, 0)),
        pl.BlockSpec((_RB, DZ), lambda r: (r, 0)),
    ],
    out_specs=[
        pl.BlockSpec((_RB, DZ), lambda r: (r, 0)),
        pl.BlockSpec((_RB, DZ), lambda r: (r, 0)),
        pl.BlockSpec((_RB, DZ), lambda r: (r, 0)),
    ],
    out_shape=[
        jax.ShapeDtypeStruct((N, DZ), jnp.float32),
        jax.ShapeDtypeStruct((N, DZ), jnp.float32),
        jax.ShapeDtypeStruct((N, DZ), jnp.float32),
    ],
)


def kernel(x, edge_index, W1, W_mu, W_logvar):
    src = edge_index[0]
    dst = edge_index[1]
    npad = EP - E
    pad_n = N + (jnp.arange(npad, dtype=jnp.int32) % (NPAD - N))
    srcp = jnp.concatenate([src, pad_n])
    dstp = jnp.concatenate([dst, pad_n])
    srca = jnp.concatenate([src, jnp.zeros((npad,), jnp.int32)])
    zeros16 = jnp.zeros((RPS, 16), jnp.float32)
    ones16 = jnp.ones((CH, 16), jnp.float32)
    zeros64 = jnp.zeros((RPS, DH), jnp.float32)

    degs = _deg_kernel(srcp, dstp, zeros16, ones16)
    y1 = _tc1(x, W1, degs)
    p1 = _agg_kernel(y1, srca, dstp, zeros64)
    y2 = _tc2(p1, degs)
    p2 = _agg_kernel(y2, srca, dstp, zeros64)
    eps = jax.random.normal(jax.random.key(42), (N, DZ), dtype=jnp.float32)
    z, mu, lv = _tc3(p2, degs, W_mu, W_logvar, eps)
    return z, mu, lv


# speedup vs baseline: 2.4335x; 1.0172x over previous
import functools

import jax
import jax.numpy as jnp
from jax import lax
from jax.experimental import pallas as pl
from jax.experimental.pallas import tpu as pltpu
from jax.experimental.pallas import tpu_sc as plsc

N = 10000
E = 320000
D = 128
DH = D // 2
DZ = 64
NC, NS = 2, 16
NW = NC * NS
CH = 128
NCH_ALL = E // CH
DEG_Q, DEG_R = NCH_ALL // NW, NCH_ALL % NW
AGG_Q, AGG_R = NCH_ALL // NS, NCH_ALL % NS
NPAD = N + 128
RPS = 624
NREM = N - RPS * NS

_sc_mesh = plsc.VectorSubcoreMesh(core_axis_name="c", subcore_axis_name="s")


@functools.partial(
    pl.kernel,
    out_type=jax.ShapeDtypeStruct((NC, 2, N, 16), jnp.float32),
    mesh=_sc_mesh,
    compiler_params=pltpu.CompilerParams(use_tc_tiling_on_sc=False),
    scratch_types=[
        pltpu.VMEM((CH,), jnp.int32),
        pltpu.VMEM((CH,), jnp.int32),
        pltpu.VMEM((CH,), jnp.int32),
        pltpu.VMEM((CH,), jnp.int32),
        pltpu.VMEM((CH, 16), jnp.float32),
        pltpu.VMEM_SHARED((NPAD, 16), jnp.float32),
        pltpu.VMEM_SHARED((NPAD, 16), jnp.float32),
        pltpu.SemaphoreType.DMA,
        pltpu.SemaphoreType.DMA,
        pltpu.SemaphoreType.DMA,
        pltpu.SemaphoreType.DMA,
    ],
)
def _deg_kernel(srcf_hbm, dstf_hbm, zeros16_hbm, ones16_hbm, out_hbm,
                idx_s0, idx_s1, idx_d0, idx_d1, ones_v, acc_s, acc_d,
                sem_l0, sem_l1, sem_c0, sem_c1):
    cid = lax.axis_index("c")
    sid = lax.axis_index("s")
    wid = cid * NS + sid
    extra = jnp.maximum(wid - (NW - DEG_R), 0)
    nch = DEG_Q + jnp.where(wid >= NW - DEG_R, 1, 0)
    base = (wid * DEG_Q + extra) * CH

    pltpu.sync_copy(ones16_hbm, ones_v)
    for acc in (acc_s, acc_d):
        pltpu.sync_copy(zeros16_hbm, acc.at[pl.ds(sid * RPS, RPS)])
        @pl.when(sid == NS - 1)
        def _(acc=acc):
            pltpu.sync_copy(zeros16_hbm.at[pl.ds(0, NREM)],
                            acc.at[pl.ds(RPS * NS, NREM)])
    plsc.subcore_barrier()

    bufs = ((idx_s0, idx_d0, sem_l0, sem_c0),
            (idx_s1, idx_d1, sem_l1, sem_c1))

    def load(c, b):
        i_s, i_d, s_l, _ = bufs[b]
        eb = base + c * CH
        pltpu.async_copy(srcf_hbm.at[pl.ds(eb, CH)], i_s, s_l)
        pltpu.async_copy(dstf_hbm.at[pl.ds(eb, CH)], i_d, s_l)

    def wait_load(b):
        i_s, i_d, s_l, _ = bufs[b]
        pltpu.make_async_copy(srcf_hbm.at[pl.ds(0, CH)], i_s, s_l).wait()
        pltpu.make_async_copy(dstf_hbm.at[pl.ds(0, CH)], i_d, s_l).wait()

    def scat(b):
        i_s, i_d, _, s_c = bufs[b]
        pltpu.async_copy(ones_v, acc_s.at[i_s], s_c, add=True)
        pltpu.async_copy(ones_v, acc_d.at[i_d], s_c, add=True)

    def wait_scat(b):
        i_s, i_d, _, s_c = bufs[b]
        pltpu.make_async_copy(ones_v, acc_s.at[i_s], s_c).wait()
        pltpu.make_async_copy(ones_v, acc_d.at[i_d], s_c).wait()

    load(0, 0)

    def body(c, _):
        for p in (0, 1):
            @pl.when(c % 2 == p)
            def _(p=p):
                wait_load(p)
                scat(p)
                @pl.when(c > 0)
                def _():
                    wait_scat(1 - p)
                @pl.when(c + 1 < nch)
                def _():
                    load(c + 1, 1 - p)
        return 0
    lax.fori_loop(0, nch, body, 0)
    @pl.when(nch % 2 == 1)
    def _():
        wait_scat(0)
    @pl.when(nch % 2 == 0)
    def _():
        wait_scat(1)
    plsc.subcore_barrier()

    sl = pl.ds(sid * RPS, RPS)
    pltpu.sync_copy(acc_s.at[sl], out_hbm.at[cid, 0, sl])
    pltpu.sync_copy(acc_d.at[sl], out_hbm.at[cid, 1, sl])
    @pl.when(sid == NS - 1)
    def _():
        rem = pl.ds(RPS * NS, NREM)
        pltpu.sync_copy(acc_s.at[rem], out_hbm.at[cid, 0, rem])
        pltpu.sync_copy(acc_d.at[rem], out_hbm.at[cid, 1, rem])


@functools.partial(
    pl.kernel,
    out_type=jax.ShapeDtypeStruct((NC, N, DH), jnp.float32),
    mesh=_sc_mesh,
    compiler_params=pltpu.CompilerParams(use_tc_tiling_on_sc=False),
    scratch_types=[
        pltpu.VMEM((CH,), jnp.int32),
        pltpu.VMEM((CH,), jnp.int32),
        pltpu.VMEM((4, CH), jnp.int32),
        pltpu.VMEM((CH, DH), jnp.float32),
        pltpu.VMEM((CH, DH), jnp.float32),
        pltpu.VMEM_SHARED((N, DH), jnp.float32),
        pltpu.VMEM_SHARED((NPAD, DH), jnp.float32),
        pltpu.SemaphoreType.DMA,
        pltpu.SemaphoreType.DMA,
        pltpu.SemaphoreType.DMA,
        pltpu.SemaphoreType.DMA,
        pltpu.SemaphoreType.DMA,
    ],
)
def _agg_kernel(y2_hbm, srcf_hbm, dstf_hbm, zeros_hbm, out_hbm,
                idx_s0, idx_s1, idx_d, rows0, rows1, ys, acc,
                sem_l0, sem_l1, sem_g0, sem_g1, sem_c):
    cid = lax.axis_index("c")
    sid = lax.axis_index("s")
    extra = jnp.maximum(sid - (NS - AGG_R), 0)
    nch = AGG_Q + jnp.where(sid >= NS - AGG_R, 1, 0)
    base = (sid * AGG_Q + extra) * CH

    sl = pl.ds(sid * RPS, RPS)
    pltpu.sync_copy(y2_hbm.at[cid, sl], ys.at[sl])
    pltpu.sync_copy(zeros_hbm, acc.at[sl])
    @pl.when(sid == NS - 1)
    def _():
        rem = pl.ds(RPS * NS, NREM)
        pltpu.sync_copy(y2_hbm.at[cid, rem], ys.at[rem])
        pltpu.sync_copy(zeros_hbm.at[pl.ds(0, NREM)], acc.at[rem])

    bufs = ((idx_s0, sem_l0, rows0, sem_g0),
            (idx_s1, sem_l1, rows1, sem_g1))

    def load(c, b):
        i_s, s_l, _, _ = bufs[b]
        eb = base + c * CH
        pltpu.async_copy(srcf_hbm.at[pl.ds(eb, CH)], i_s, s_l)
        pltpu.async_copy(dstf_hbm.at[pl.ds(eb, CH)], idx_d.at[c % 4], s_l)

    def wait_load(b):
        i_s, s_l, _, _ = bufs[b]
        pltpu.make_async_copy(srcf_hbm.at[pl.ds(0, CH)], i_s, s_l).wait()
        pltpu.make_async_copy(srcf_hbm.at[pl.ds(0, CH)], i_s, s_l).wait()

    def gather(b):
        i_s, _, r, s_g = bufs[b]
        pltpu.async_copy(ys.at[i_s], r, s_g)

    def wait_gather(b):
        i_s, _, r, s_g = bufs[b]
        pltpu.make_async_copy(ys.at[i_s], r, s_g).wait()

    def scat(c, b):
        _, _, r, _ = bufs[b]
        pltpu.async_copy(r, acc.at[idx_d.at[c % 4]], sem_c, add=True)

    def wait_scat(b):
        _, _, r, _ = bufs[b]
        pltpu.make_async_copy(r, acc.at[idx_d.at[0]], sem_c).wait()

    load(0, 0)
    wait_load(0)
    load(1, 1)
    plsc.subcore_barrier()
    gather(0)

    def body(c, _):
        for p in (0, 1):
            @pl.when(c % 2 == p)
            def _(p=p):
                @pl.when(c + 1 < nch)
                def _():
                    wait_load(1 - p)
                @pl.when(c > 0)
                def _():
                    wait_scat(1 - p)
                @pl.when(c + 1 < nch)
                def _():
                    gather(1 - p)
                wait_gather(p)
                scat(c, p)
                @pl.when(c + 2 < nch)
                def _():
                    load(c + 2, p)
        return 0
    lax.fori_loop(0, nch, body, 0)
    @pl.when(nch % 2 == 1)
    def _():
        wait_scat(0)
    @pl.when(nch % 2 == 0)
    def _():
        wait_scat(1)
    plsc.subcore_barrier()

    pltpu.sync_copy(acc.at[sl], out_hbm.at[cid, sl])
    @pl.when(sid == NS - 1)
    def _():
        rem = pl.ds(RPS * NS, NREM)
        pltpu.sync_copy(acc.at[rem], out_hbm.at[cid, rem])


_RB = 1000
_GRID = N // _RB

_deg_spec = pl.BlockSpec((NC, 2, _RB, 16), lambda r: (0, 0, r, 0))
_half_spec = pl.BlockSpec((2, _RB, DH), lambda r: (0, r, 0))


def _norms(degref):
    deg_out = degref[0, 0, :, :1] + degref[1, 0, :, :1]
    deg_in = degref[0, 1, :, :1] + degref[1, 1, :, :1]
    n_out = lax.rsqrt(jnp.maximum(deg_out, 1.0))
    n_in = lax.rsqrt(jnp.maximum(deg_in, 1.0))
    return n_out, n_in


def _tc1_body(x_ref, w_ref, deg_ref, y_ref):
    n_out, _ = _norms(deg_ref)
    xv = x_ref[...]
    y_ref[0] = jnp.dot(xv, w_ref[:, :DH],
                       preferred_element_type=jnp.float32) * n_out
    y_ref[1] = jnp.dot(xv, w_ref[:, DH:],
                       preferred_element_type=jnp.float32) * n_out


_tc1 = pl.pallas_call(
    _tc1_body,
    grid=(_GRID,),
    in_specs=[
        pl.BlockSpec((_RB, D), lambda r: (r, 0)),
        pl.BlockSpec((D, D), lambda r: (0, 0)),
        _deg_spec,
    ],
    out_specs=_half_spec,
    out_shape=jax.ShapeDtypeStruct((2, N, DH), jnp.float32),
)


def _tc2_body(p_ref, deg_ref, y_ref):
    n_out, n_in = _norms(deg_ref)
    for i in range(2):
        h = jnp.maximum(p_ref[i] * n_in, 0.0)
        y_ref[i] = h * n_out


_tc2 = pl.pallas_call(
    _tc2_body,
    grid=(_GRID,),
    in_specs=[_half_spec, _deg_spec],
    out_specs=_half_spec,
    out_shape=jax.ShapeDtypeStruct((2, N, DH), jnp.float32),
)


def _tc3_body(p_ref, deg_ref, wmu_ref, wlv_ref, eps_ref, z_ref, mu_ref,
              lv_ref):
    _, n_in = _norms(deg_ref)
    a_lo = p_ref[0] * n_in
    a_hi = p_ref[1] * n_in
    mu = (jnp.dot(a_lo, wmu_ref[:DH, :], preferred_element_type=jnp.float32)
          + jnp.dot(a_hi, wmu_ref[DH:, :], preferred_element_type=jnp.float32))
    lv = (jnp.dot(a_lo, wlv_ref[:DH, :], preferred_element_type=jnp.float32)
          + jnp.dot(a_hi, wlv_ref[DH:, :], preferred_element_type=jnp.float32))
    mu_ref[...] = mu
    lv_ref[...] = lv
    z_ref[...] = mu + jnp.exp(0.5 * lv) * eps_ref[...]


_tc3 = pl.pallas_call(
    _tc3_body,
    grid=(_GRID,),
    in_specs=[
        _half_spec,
        _deg_spec,
        pl.BlockSpec((D, DZ), lambda r: (0, 0)),
        pl.BlockSpec((D, DZ), lambda r: (0, 0)),
        pl.BlockSpec((_RB, DZ), lambda r: (r, 0)),
    ],
    out_specs=[
        pl.BlockSpec((_RB, DZ), lambda r: (r, 0)),
        pl.BlockSpec((_RB, DZ), lambda r: (r, 0)),
        pl.BlockSpec((_RB, DZ), lambda r: (r, 0)),
    ],
    out_shape=[
        jax.ShapeDtypeStruct((N, DZ), jnp.float32),
        jax.ShapeDtypeStruct((N, DZ), jnp.float32),
        jax.ShapeDtypeStruct((N, DZ), jnp.float32),
    ],
)


def kernel(x, edge_index, W1, W_mu, W_logvar):
    src = edge_index[0]
    dst = edge_index[1]
    zeros16 = jnp.zeros((RPS, 16), jnp.float32)
    ones16 = jnp.ones((CH, 16), jnp.float32)
    zeros64 = jnp.zeros((RPS, DH), jnp.float32)

    degs = _deg_kernel(src, dst, zeros16, ones16)
    y1 = _tc1(x, W1, degs)
    p1 = _agg_kernel(y1, src, dst, zeros64)
    y2 = _tc2(p1, degs)
    p2 = _agg_kernel(y2, src, dst, zeros64)
    eps = jax.random.normal(jax.random.key(42), (N, DZ), dtype=jnp.float32)
    z, mu, lv = _tc3(p2, degs, W_mu, W_logvar, eps)
    return z, mu, lv
